# baseline (speedup 1.0000x reference)
import jax
import jax.numpy as jnp
from jax import lax
from jax.experimental import pallas as pl
from jax.experimental.pallas import tpu as pltpu
from jax.experimental.pallas import tpu_sc as plsc

B, N, M = 4, 8192, 1024
C_REF, C_SKIP, C_T = 256, 128, 128
NB = 512

NC, NS, L = 2, 16, 16
NW = NC * NS
CPG = 8
NGF = C_REF // CPG
NGT = C_T // CPG
NH = N // 2
CH = 1024


def _nn_body(ct_ref, rc_ref, i0_ref, i1_ref, i2_ref, w0_ref, w1_ref, w2_ref):
    ct = ct_ref[0]
    rc = rc_ref[0]

    dx = ct[0:1, :] - rc[:, 0:1]
    dy = ct[1:2, :] - rc[:, 1:2]
    dz = ct[2:3, :] - rc[:, 2:3]
    d2 = dx * dx + dy * dy + dz * dz

    iota_m = jax.lax.broadcasted_iota(jnp.int32, (M, NB), 0)
    idxs = []
    dists = []
    for k in range(3):
        dk = jnp.min(d2, axis=0, keepdims=True)
        ik = jnp.min(jnp.where(d2 == dk, iota_m, M), axis=0, keepdims=True)
        idxs.append(ik)
        dists.append(dk)
        if k < 2:
            d2 = jnp.where(iota_m == ik, jnp.float32(jnp.inf), d2)

    w0 = 1.0 / jnp.maximum(dists[0], 1e-10)
    w1 = 1.0 / jnp.maximum(dists[1], 1e-10)
    w2 = 1.0 / jnp.maximum(dists[2], 1e-10)
    s = w0 + w1 + w2

    i0_ref[...] = jnp.reshape(idxs[0], (NB,))
    i1_ref[...] = jnp.reshape(idxs[1], (NB,))
    i2_ref[...] = jnp.reshape(idxs[2], (NB,))
    w0_ref[...] = jnp.reshape(w0 / s, (NB,))
    w1_ref[...] = jnp.reshape(w1 / s, (NB,))
    w2_ref[...] = jnp.reshape(w2 / s, (NB,))


def _three_nn(coords_t, ref_coords):
    flat = pl.BlockSpec((NB,), lambda b, n: (b * (N // NB) + n,))
    return pl.pallas_call(
        _nn_body,
        grid=(B, N // NB),
        in_specs=[
            pl.BlockSpec((1, 3, NB), lambda b, n: (b, 0, n)),
            pl.BlockSpec((1, M, 3), lambda b, n: (b, 0, 0)),
        ],
        out_specs=[flat] * 6,
        out_shape=[jax.ShapeDtypeStruct((B * N,), jnp.int32)] * 3
        + [jax.ShapeDtypeStruct((B * N,), jnp.float32)] * 3,
    )(coords_t, ref_coords)


def _interp_body(i0_hbm, i1_hbm, i2_hbm, w0_hbm, w1_hbm, w2_hbm,
                 rf_hbm, rt_hbm, pf_hbm, feat_hbm, temb_hbm,
                 tab, ib0, ib1, ib2, wb0, wb1, wb2, obuf):
    wid = lax.axis_index("s") * NC + lax.axis_index("c")

    for b in range(B):
        pltpu.sync_copy(i0_hbm.at[pl.ds(b * N, N)], ib0)
        pltpu.sync_copy(i1_hbm.at[pl.ds(b * N, N)], ib1)
        pltpu.sync_copy(i2_hbm.at[pl.ds(b * N, N)], ib2)
        pltpu.sync_copy(w0_hbm.at[pl.ds(b * N, N)], wb0)
        pltpu.sync_copy(w1_hbm.at[pl.ds(b * N, N)], wb1)
        pltpu.sync_copy(w2_hbm.at[pl.ds(b * N, N)], wb2)

        def run_unit(table_hbm, out_hbm, row0, base_n):
            pltpu.sync_copy(table_hbm.at[b, pl.ds(row0, CPG), :], tab)

            def chunk_body(ci, carry):
                n0 = base_n + ci * CH

                def group_body(g, carry2):
                    off = n0 + g * L
                    i0 = ib0[pl.ds(off, L)]
                    i1 = ib1[pl.ds(off, L)]
                    i2 = ib2[pl.ds(off, L)]
                    w0 = wb0[pl.ds(off, L)]
                    w1 = wb1[pl.ds(off, L)]
                    w2 = wb2[pl.ds(off, L)]
                    for c in range(CPG):
                        row = jnp.full((L,), c, jnp.int32)
                        v = plsc.load_gather(tab, [row, i0]) * w0
                        v = v + plsc.load_gather(tab, [row, i1]) * w1
                        v = v + plsc.load_gather(tab, [row, i2]) * w2
                        obuf[c, pl.ds(g * L, L)] = v
                    return carry2

                lax.fori_loop(0, CH // L, group_body, 0)
                pltpu.sync_copy(obuf, out_hbm.at[b, pl.ds(row0, CPG), pl.ds(n0, CH)])
                return carry

            lax.fori_loop(0, NH // CH, chunk_body, 0)

        for u3 in range(3):
            unit = wid * 3 + u3
            cg = unit // 2
            base_n = (unit % 2) * NH
            lax.cond(
                cg < NGF,
                lambda cg=cg, base_n=base_n: run_unit(rf_hbm, feat_hbm, cg * CPG, base_n),
                lambda cg=cg, base_n=base_n: run_unit(
                    rt_hbm, temb_hbm, (cg - NGF) * CPG, base_n
                ),
            )

        srow = (wid // 2) * CPG
        sbase = (wid % 2) * NH

        def skip_body(ci, carry):
            n0 = sbase + ci * CH
            pltpu.sync_copy(pf_hbm.at[b, pl.ds(srow, CPG), pl.ds(n0, CH)], obuf)
            pltpu.sync_copy(obuf, feat_hbm.at[b, pl.ds(C_REF + srow, CPG), pl.ds(n0, CH)])
            return carry

        lax.fori_loop(0, NH // CH, skip_body, 0)


_interp = pl.kernel(
    _interp_body,
    out_type=[
        jax.ShapeDtypeStruct((B, C_REF + C_SKIP, N), jnp.float32),
        jax.ShapeDtypeStruct((B, C_T, N), jnp.float32),
    ],
    mesh=plsc.VectorSubcoreMesh(
        core_axis_name="c", subcore_axis_name="s", num_cores=NC, num_subcores=NS
    ),
    compiler_params=pltpu.CompilerParams(
        use_tc_tiling_on_sc=False, needs_layout_passes=False
    ),
    scratch_types=[
        pltpu.VMEM((CPG, M), jnp.float32),
        pltpu.VMEM((N,), jnp.int32),
        pltpu.VMEM((N,), jnp.int32),
        pltpu.VMEM((N,), jnp.int32),
        pltpu.VMEM((N,), jnp.float32),
        pltpu.VMEM((N,), jnp.float32),
        pltpu.VMEM((N,), jnp.float32),
        pltpu.VMEM((CPG, CH), jnp.float32),
    ],
)


def kernel(points_coords, points_features, ref_coords, ref_features, ref_t_embed):
    coords_t = jnp.transpose(points_coords, (0, 2, 1))
    i0, i1, i2, w0, w1, w2 = _three_nn(coords_t, ref_coords)
    out = _interp(i0, i1, i2, w0, w1, w2, ref_features, ref_t_embed, points_features)
    return out[0], out[1]


# speedup vs baseline: 1.9016x; 1.9016x over previous
import functools

import jax
import jax.numpy as jnp
from jax import lax
from jax.experimental import pallas as pl
from jax.experimental.pallas import tpu as pltpu
from jax.experimental.pallas import tpu_sc as plsc

B, N, M = 4, 8192, 1024
C_REF, C_SKIP, C_T = 256, 128, 128
NB = 512

NC, NS, L = 2, 16, 16
NW = NC * NS
CPG = 8
NGF = C_REF // CPG
NGT = C_T // CPG
NH = N // 2
CH = 2048


def _nn_body(ct_ref, rc_ref, i0_ref, i1_ref, i2_ref, w0_ref, w1_ref, w2_ref):
    ct = ct_ref[0]
    rc = rc_ref[0]

    dx = ct[0:1, :] - rc[:, 0:1]
    dy = ct[1:2, :] - rc[:, 1:2]
    dz = ct[2:3, :] - rc[:, 2:3]
    d2 = dx * dx + dy * dy + dz * dz

    iota_f = jax.lax.broadcasted_iota(jnp.int32, (M, NB), 0).astype(jnp.float32)
    idxs = []
    dists = []
    for k in range(3):
        dk = jnp.min(d2, axis=0, keepdims=True)
        ik = jnp.min(
            jnp.where(d2 == dk, iota_f, jnp.float32(M)), axis=0, keepdims=True
        )
        idxs.append(ik.astype(jnp.int32))
        dists.append(dk)
        if k < 2:
            d2 = jnp.where(iota_f == ik, jnp.float32(jnp.inf), d2)

    w0 = 1.0 / jnp.maximum(dists[0], 1e-10)
    w1 = 1.0 / jnp.maximum(dists[1], 1e-10)
    w2 = 1.0 / jnp.maximum(dists[2], 1e-10)
    s = w0 + w1 + w2

    i0_ref[...] = jnp.reshape(idxs[0], (NB,))
    i1_ref[...] = jnp.reshape(idxs[1], (NB,))
    i2_ref[...] = jnp.reshape(idxs[2], (NB,))
    w0_ref[...] = jnp.reshape(w0 / s, (NB,))
    w1_ref[...] = jnp.reshape(w1 / s, (NB,))
    w2_ref[...] = jnp.reshape(w2 / s, (NB,))


def _three_nn(coords_t, ref_coords, nb):
    flat = pl.BlockSpec((NB,), lambda b, n: (b * (N // NB) + n,))
    return pl.pallas_call(
        _nn_body,
        grid=(nb, N // NB),
        in_specs=[
            pl.BlockSpec((1, 3, NB), lambda b, n: (b, 0, n)),
            pl.BlockSpec((1, M, 3), lambda b, n: (b, 0, 0)),
        ],
        out_specs=[flat] * 6,
        out_shape=[jax.ShapeDtypeStruct((nb * N,), jnp.int32)] * 3
        + [jax.ShapeDtypeStruct((nb * N,), jnp.float32)] * 3,
    )(coords_t, ref_coords)


def _interp_body(nb, i0_hbm, i1_hbm, i2_hbm, w0_hbm, w1_hbm, w2_hbm,
                 rf_hbm, rt_hbm, feat_hbm, temb_hbm,
                 tab, ib0, ib1, ib2, wb0, wb1, wb2, oba, obb,
                 sem_i, sem_a, sem_b):
    wid = lax.axis_index("s") * NC + lax.axis_index("c")
    obufs = (oba, obb)
    sems = (sem_a, sem_b)

    def batch_body(b, carry):
        descs = [
            pltpu.async_copy(src.at[pl.ds(b * N, N)], dst, sem_i)
            for src, dst in (
                (i0_hbm, ib0), (i1_hbm, ib1), (i2_hbm, ib2),
                (w0_hbm, wb0), (w1_hbm, wb1), (w2_hbm, wb2),
            )
        ]
        for u3 in range(3):
            cg = (wid * 3 + u3) // 2
            tslice = tab.at[pl.ds(u3 * CPG * M, CPG * M)]

            def stage_feat(cg=cg, tslice=tslice):
                pltpu.async_copy(
                    rf_hbm.at[pl.ds((b * C_REF + cg * CPG) * M, CPG * M)],
                    tslice, sem_i,
                )

            def stage_t(cg=cg, tslice=tslice):
                pltpu.async_copy(
                    rt_hbm.at[pl.ds((b * C_T + (cg - NGF) * CPG) * M, CPG * M)],
                    tslice, sem_i,
                )

            lax.cond(cg < NGF, stage_feat, stage_t)
        for d in descs:
            d.wait()
        for u3 in range(3):
            pltpu.make_async_copy(
                rf_hbm.at[pl.ds(0, CPG * M)],
                tab.at[pl.ds(u3 * CPG * M, CPG * M)], sem_i,
            ).wait()

        def run_chunks(out_hbm, row0, base_n, tab_base):
            pending = [False, False]
            for ci in range(NH // CH):
                buf = ci % 2
                ob = obufs[buf]
                n0 = base_n + ci * CH
                if pending[buf]:
                    pltpu.make_async_copy(
                        ob, out_hbm.at[b, pl.ds(row0, CPG), pl.ds(n0, CH)], sems[buf]
                    ).wait()
                    pending[buf] = False

                @plsc.parallel_loop(0, CH // L, 1, unroll=2)
                def group_body(g):
                    off = n0 + g * L
                    i0 = ib0[pl.ds(off, L)]
                    i1 = ib1[pl.ds(off, L)]
                    i2 = ib2[pl.ds(off, L)]
                    w0 = wb0[pl.ds(off, L)]
                    w1 = wb1[pl.ds(off, L)]
                    w2 = wb2[pl.ds(off, L)]
                    a0 = i0 + tab_base
                    a1 = i1 + tab_base
                    a2 = i2 + tab_base
                    for c in range(CPG):
                        v = plsc.load_gather(tab, [a0]) * w0
                        v = v + plsc.load_gather(tab, [a1]) * w1
                        v = v + plsc.load_gather(tab, [a2]) * w2
                        ob[c, pl.ds(g * L, L)] = v
                        if c + 1 < CPG:
                            a0 = a0 + M
                            a1 = a1 + M
                            a2 = a2 + M

                pltpu.async_copy(
                    ob, out_hbm.at[b, pl.ds(row0, CPG), pl.ds(n0, CH)], sems[buf]
                )
                pending[buf] = True
            for buf in range(2):
                if pending[buf]:
                    pltpu.make_async_copy(
                        obufs[buf],
                        out_hbm.at[b, pl.ds(row0, CPG), pl.ds(base_n, CH)],
                        sems[buf],
                    ).wait()

        def unit_body(u3, carry2):
            unit = wid * 3 + u3
            cg = unit // 2
            base_n = (unit % 2) * NH
            tab_base = u3 * (CPG * M)

            def chunks_feat():
                run_chunks(feat_hbm, cg * CPG, base_n, tab_base)

            def chunks_t():
                run_chunks(temb_hbm, (cg - NGF) * CPG, base_n, tab_base)

            lax.cond(cg < NGF, chunks_feat, chunks_t)
            return carry2

        lax.fori_loop(0, 3, unit_body, 0)
        return carry

    lax.fori_loop(0, nb, batch_body, 0)


def _make_interp(nb):
    return pl.kernel(
        functools.partial(_interp_body, nb),
        out_type=[
            jax.ShapeDtypeStruct((nb, C_REF, N), jnp.float32),
            jax.ShapeDtypeStruct((nb, C_T, N), jnp.float32),
        ],
    mesh=plsc.VectorSubcoreMesh(
        core_axis_name="c", subcore_axis_name="s", num_cores=NC, num_subcores=NS
    ),
    compiler_params=pltpu.CompilerParams(
        use_tc_tiling_on_sc=False, needs_layout_passes=False
    ),
    scratch_types=[
        pltpu.VMEM((3 * CPG * M,), jnp.float32),
        pltpu.VMEM((N,), jnp.int32),
        pltpu.VMEM((N,), jnp.int32),
        pltpu.VMEM((N,), jnp.int32),
        pltpu.VMEM((N,), jnp.float32),
        pltpu.VMEM((N,), jnp.float32),
        pltpu.VMEM((N,), jnp.float32),
        pltpu.VMEM((CPG, CH), jnp.float32),
        pltpu.VMEM((CPG, CH), jnp.float32),
        pltpu.SemaphoreType.DMA,
        pltpu.SemaphoreType.DMA,
        pltpu.SemaphoreType.DMA,
    ],
    )


_interp1 = _make_interp(1)


def kernel(points_coords, points_features, ref_coords, ref_features, ref_t_embed):
    coords_t = jnp.transpose(points_coords, (0, 2, 1))
    nn = [
        _three_nn(coords_t[b : b + 1], ref_coords[b : b + 1], 1) for b in range(B)
    ]
    feats = []
    tembs = []
    for b in range(B):
        rf_b = jnp.reshape(ref_features[b], (C_REF * M,))
        rt_b = jnp.reshape(ref_t_embed[b], (C_T * M,))
        o = _interp1(*nn[b], rf_b, rt_b)
        feats.append(o[0])
        tembs.append(o[1])
    features = jnp.concatenate(
        [jnp.concatenate(feats, axis=0), points_features], axis=1
    )
    return features, jnp.concatenate(tembs, axis=0)


# speedup vs baseline: 1.9148x; 1.0070x over previous
import functools

import jax
import jax.numpy as jnp
from jax import lax
from jax.experimental import pallas as pl
from jax.experimental.pallas import tpu as pltpu
from jax.experimental.pallas import tpu_sc as plsc

B, N, M = 4, 8192, 1024
C_REF, C_SKIP, C_T = 256, 128, 128
NB = 512

NC, NS, L = 2, 16, 16
NW = NC * NS
CPG = 8
NGF = C_REF // CPG
NGT = C_T // CPG
NH = N // 2
CH = 2048


CK = 64


def _nn_body(ct_ref, rc_ref, i0_ref, i1_ref, i2_ref, w0_ref, w1_ref, w2_ref):
    ct = ct_ref[0]
    rc = rc_ref[0]

    px = ct[0:1, :]
    py = ct[1:2, :]
    pz = ct[2:3, :]
    iota_l = jax.lax.broadcasted_iota(jnp.int32, (CK, NB), 0)
    inf = jnp.float32(jnp.inf)
    r1 = r2 = r3 = jnp.full((1, NB), inf, jnp.float32)
    for j in range(M // CK):
        rcj = rc[j * CK:(j + 1) * CK]
        dx = px - rcj[:, 0:1]
        dy = py - rcj[:, 1:2]
        dz = pz - rcj[:, 2:3]
        d2 = dx * dx + dy * dy + dz * dz
        ki = jax.lax.bitwise_and(
            jax.lax.bitcast_convert_type(d2, jnp.int32) + 512, jnp.int32(-1024)
        )
        key = jax.lax.bitcast_convert_type(
            jax.lax.bitwise_or(ki, iota_l + (j * CK)), jnp.float32
        )
        c1 = jnp.min(key, axis=0, keepdims=True)
        k2 = jnp.where(key == c1, inf, key)
        c2 = jnp.min(k2, axis=0, keepdims=True)
        c3 = jnp.min(jnp.where(k2 == c2, inf, k2), axis=0, keepdims=True)
        x = jnp.maximum(r1, c1)
        m1 = jnp.minimum(r1, c1)
        y = jnp.minimum(r2, c2)
        z = jnp.maximum(r2, c2)
        w_ = jnp.minimum(r3, c3)
        r1 = m1
        r3 = jnp.minimum(jnp.maximum(x, y), jnp.minimum(z, w_))
        r2 = jnp.minimum(x, y)

    idxs = []
    dists = []
    for rk in (r1, r2, r3):
        rki = jax.lax.bitcast_convert_type(rk, jnp.int32)
        idxs.append(jax.lax.bitwise_and(rki, jnp.int32(1023)))
        dists.append(
            jax.lax.bitcast_convert_type(
                jax.lax.bitwise_and(rki, jnp.int32(-1024)), jnp.float32
            )
        )

    w0 = 1.0 / jnp.maximum(dists[0], 1e-10)
    w1 = 1.0 / jnp.maximum(dists[1], 1e-10)
    w2 = 1.0 / jnp.maximum(dists[2], 1e-10)
    s = w0 + w1 + w2

    i0_ref[...] = jnp.reshape(idxs[0], (NB,))
    i1_ref[...] = jnp.reshape(idxs[1], (NB,))
    i2_ref[...] = jnp.reshape(idxs[2], (NB,))
    w0_ref[...] = jnp.reshape(w0 / s, (NB,))
    w1_ref[...] = jnp.reshape(w1 / s, (NB,))
    w2_ref[...] = jnp.reshape(w2 / s, (NB,))


def _three_nn(coords_t, ref_coords, nb):
    flat = pl.BlockSpec((NB,), lambda b, n: (b * (N // NB) + n,))
    return pl.pallas_call(
        _nn_body,
        grid=(nb, N // NB),
        in_specs=[
            pl.BlockSpec((1, 3, NB), lambda b, n: (b, 0, n)),
            pl.BlockSpec((1, M, 3), lambda b, n: (b, 0, 0)),
        ],
        out_specs=[flat] * 6,
        out_shape=[jax.ShapeDtypeStruct((nb * N,), jnp.int32)] * 3
        + [jax.ShapeDtypeStruct((nb * N,), jnp.float32)] * 3,
    )(coords_t, ref_coords)


def _interp_body(nb, i0_hbm, i1_hbm, i2_hbm, w0_hbm, w1_hbm, w2_hbm,
                 rf_hbm, rt_hbm, feat_hbm, temb_hbm,
                 tab, ib0, ib1, ib2, wb0, wb1, wb2, oba, obb,
                 sem_i, sem_a, sem_b):
    wid = lax.axis_index("s") * NC + lax.axis_index("c")
    obufs = (oba, obb)
    sems = (sem_a, sem_b)

    def batch_body(b, carry):
        descs = [
            pltpu.async_copy(src.at[pl.ds(b * N, N)], dst, sem_i)
            for src, dst in (
                (i0_hbm, ib0), (i1_hbm, ib1), (i2_hbm, ib2),
                (w0_hbm, wb0), (w1_hbm, wb1), (w2_hbm, wb2),
            )
        ]
        for u3 in range(3):
            cg = (wid * 3 + u3) // 2
            tslice = tab.at[pl.ds(u3 * CPG * M, CPG * M)]

            def stage_feat(cg=cg, tslice=tslice):
                pltpu.async_copy(
                    rf_hbm.at[pl.ds((b * C_REF + cg * CPG) * M, CPG * M)],
                    tslice, sem_i,
                )

            def stage_t(cg=cg, tslice=tslice):
                pltpu.async_copy(
                    rt_hbm.at[pl.ds((b * C_T + (cg - NGF) * CPG) * M, CPG * M)],
                    tslice, sem_i,
                )

            lax.cond(cg < NGF, stage_feat, stage_t)
        for d in descs:
            d.wait()
        for u3 in range(3):
            pltpu.make_async_copy(
                rf_hbm.at[pl.ds(0, CPG * M)],
                tab.at[pl.ds(u3 * CPG * M, CPG * M)], sem_i,
            ).wait()

        def run_chunks(out_hbm, row0, base_n, tab_base):
            pending = [False, False]
            for ci in range(NH // CH):
                buf = ci % 2
                ob = obufs[buf]
                n0 = base_n + ci * CH
                if pending[buf]:
                    pltpu.make_async_copy(
                        ob, out_hbm.at[b, pl.ds(row0, CPG), pl.ds(n0, CH)], sems[buf]
                    ).wait()
                    pending[buf] = False

                @plsc.parallel_loop(0, CH // L, 1, unroll=2)
                def group_body(g):
                    off = n0 + g * L
                    i0 = ib0[pl.ds(off, L)]
                    i1 = ib1[pl.ds(off, L)]
                    i2 = ib2[pl.ds(off, L)]
                    w0 = wb0[pl.ds(off, L)]
                    w1 = wb1[pl.ds(off, L)]
                    w2 = wb2[pl.ds(off, L)]
                    a0 = i0 + tab_base
                    a1 = i1 + tab_base
                    a2 = i2 + tab_base
                    for c in range(CPG):
                        v = plsc.load_gather(tab, [a0]) * w0
                        v = v + plsc.load_gather(tab, [a1]) * w1
                        v = v + plsc.load_gather(tab, [a2]) * w2
                        ob[c, pl.ds(g * L, L)] = v
                        if c + 1 < CPG:
                            a0 = a0 + M
                            a1 = a1 + M
                            a2 = a2 + M

                pltpu.async_copy(
                    ob, out_hbm.at[b, pl.ds(row0, CPG), pl.ds(n0, CH)], sems[buf]
                )
                pending[buf] = True
            for buf in range(2):
                if pending[buf]:
                    pltpu.make_async_copy(
                        obufs[buf],
                        out_hbm.at[b, pl.ds(row0, CPG), pl.ds(base_n, CH)],
                        sems[buf],
                    ).wait()

        def unit_body(u3, carry2):
            unit = wid * 3 + u3
            cg = unit // 2
            base_n = (unit % 2) * NH
            tab_base = u3 * (CPG * M)

            def chunks_feat():
                run_chunks(feat_hbm, cg * CPG, base_n, tab_base)

            def chunks_t():
                run_chunks(temb_hbm, (cg - NGF) * CPG, base_n, tab_base)

            lax.cond(cg < NGF, chunks_feat, chunks_t)
            return carry2

        lax.fori_loop(0, 3, unit_body, 0)
        return carry

    lax.fori_loop(0, nb, batch_body, 0)


def _make_interp(nb):
    return pl.kernel(
        functools.partial(_interp_body, nb),
        out_type=[
            jax.ShapeDtypeStruct((nb, C_REF, N), jnp.float32),
            jax.ShapeDtypeStruct((nb, C_T, N), jnp.float32),
        ],
    mesh=plsc.VectorSubcoreMesh(
        core_axis_name="c", subcore_axis_name="s", num_cores=NC, num_subcores=NS
    ),
    compiler_params=pltpu.CompilerParams(
        use_tc_tiling_on_sc=False, needs_layout_passes=False
    ),
    scratch_types=[
        pltpu.VMEM((3 * CPG * M,), jnp.float32),
        pltpu.VMEM((N,), jnp.int32),
        pltpu.VMEM((N,), jnp.int32),
        pltpu.VMEM((N,), jnp.int32),
        pltpu.VMEM((N,), jnp.float32),
        pltpu.VMEM((N,), jnp.float32),
        pltpu.VMEM((N,), jnp.float32),
        pltpu.VMEM((CPG, CH), jnp.float32),
        pltpu.VMEM((CPG, CH), jnp.float32),
        pltpu.SemaphoreType.DMA,
        pltpu.SemaphoreType.DMA,
        pltpu.SemaphoreType.DMA,
    ],
    )


_interp1 = _make_interp(1)


def kernel(points_coords, points_features, ref_coords, ref_features, ref_t_embed):
    coords_t = jnp.transpose(points_coords, (0, 2, 1))
    nn = [
        _three_nn(coords_t[b : b + 1], ref_coords[b : b + 1], 1) for b in range(B)
    ]
    feats = []
    tembs = []
    for b in range(B):
        rf_b = jnp.reshape(ref_features[b], (C_REF * M,))
        rt_b = jnp.reshape(ref_t_embed[b], (C_T * M,))
        o = _interp1(*nn[b], rf_b, rt_b)
        feats.append(o[0])
        tembs.append(o[1])
    features = jnp.concatenate(
        [jnp.concatenate(feats, axis=0), points_features], axis=1
    )
    return features, jnp.concatenate(tembs, axis=0)
